# Initial kernel scaffold; baseline (speedup 1.0000x reference)
import functools

import jax
import jax.numpy as jnp
from jax import lax
from jax.experimental import pallas as pl
from jax.experimental.pallas import tpu as pltpu
from jax.experimental.pallas import tpu_sc as plsc

_B = 4096
_F = 26
_VOCAB = 100000
_DIM = 16
_L1, _L2, _L3 = 512, 256, 128

_NC = 2
_NS = 16
_NW = _NC * _NS
_IDX_TOTAL = _B * _F
_IDX_PER_W = _IDX_TOTAL // _NW
_CHUNK = 128
_NCHUNK = _IDX_PER_W // _CHUNK


def _sc_body(idx_hbm, embed_hbm, wide_hbm, emb_out, wide_out,
             idx_v, emb_v, wide_v, sem_e, sem_w):
    wid = lax.axis_index("s") * _NC + lax.axis_index("c")
    base = wid * _IDX_PER_W
    pltpu.sync_copy(idx_hbm.at[pl.ds(wid * _NCHUNK, _NCHUNK)], idx_v)

    def fire(j, carry):
        pltpu.async_copy(embed_hbm.at[idx_v.at[j]],
                         emb_v.at[pl.ds(j * _CHUNK, _CHUNK)], sem_e)
        pltpu.async_copy(wide_hbm.at[idx_v.at[j]],
                         wide_v.at[pl.ds(j * _CHUNK, _CHUNK)], sem_w)
        return carry

    lax.fori_loop(0, _NCHUNK, fire, 0)

    def drain(j, carry):
        pltpu.make_async_copy(embed_hbm.at[idx_v.at[j]],
                              emb_v.at[pl.ds(j * _CHUNK, _CHUNK)], sem_e).wait()
        pltpu.make_async_copy(wide_hbm.at[idx_v.at[j]],
                              wide_v.at[pl.ds(j * _CHUNK, _CHUNK)], sem_w).wait()
        return carry

    lax.fori_loop(0, _NCHUNK, drain, 0)

    pltpu.sync_copy(emb_v, emb_out.at[pl.ds(base, _IDX_PER_W)])
    pltpu.sync_copy(wide_v, wide_out.at[pl.ds(base, _IDX_PER_W)])


_sc_gather = functools.partial(
    pl.kernel,
    out_type=[
        jax.ShapeDtypeStruct((_IDX_TOTAL, _DIM), jnp.float32),
        jax.ShapeDtypeStruct((_IDX_TOTAL, 1), jnp.float32),
    ],
    mesh=plsc.VectorSubcoreMesh(core_axis_name="c", subcore_axis_name="s"),
    scratch_types=[
        pltpu.VMEM((_NCHUNK, _CHUNK), jnp.int32),
        pltpu.VMEM((_IDX_PER_W, _DIM), jnp.float32),
        pltpu.VMEM((_IDX_PER_W, 1), jnp.float32),
        pltpu.SemaphoreType.DMA,
        pltpu.SemaphoreType.DMA,
    ],
)()(_sc_body)


_BLK = 512


def _mlp_body(deep_ref, widev_ref, w1_ref, b1_ref, w2_ref, b2_ref,
              w3_ref, b3_ref, woutt_ref, bias_ref, out_ref):
    x = deep_ref[...]
    h = jnp.maximum(jnp.dot(x, w1_ref[...], preferred_element_type=jnp.float32)
                    + b1_ref[...], 0.0)
    h = jnp.maximum(jnp.dot(h, w2_ref[...], preferred_element_type=jnp.float32)
                    + b2_ref[...], 0.0)
    h = jnp.maximum(jnp.dot(h, w3_ref[...], preferred_element_type=jnp.float32)
                    + b3_ref[...], 0.0)
    deep = jnp.sum(h * woutt_ref[...], axis=1, keepdims=True)
    wide = jnp.sum(widev_ref[...], axis=1, keepdims=True)
    out_ref[...] = deep + wide + bias_ref[0, 0]


def _mlp(deep_in, widev, w1, b1, w2, b2, w3, b3, woutt, bias):
    return pl.pallas_call(
        _mlp_body,
        grid=(_B // _BLK,),
        in_specs=[
            pl.BlockSpec((_BLK, _F * _DIM), lambda i: (i, 0)),
            pl.BlockSpec((_BLK, _F), lambda i: (i, 0)),
            pl.BlockSpec((_F * _DIM, _L1), lambda i: (0, 0)),
            pl.BlockSpec((1, _L1), lambda i: (0, 0)),
            pl.BlockSpec((_L1, _L2), lambda i: (0, 0)),
            pl.BlockSpec((1, _L2), lambda i: (0, 0)),
            pl.BlockSpec((_L2, _L3), lambda i: (0, 0)),
            pl.BlockSpec((1, _L3), lambda i: (0, 0)),
            pl.BlockSpec((1, _L3), lambda i: (0, 0)),
            pl.BlockSpec((1, 1), lambda i: (0, 0)),
        ],
        out_specs=pl.BlockSpec((_BLK, 1), lambda i: (i, 0)),
        out_shape=jax.ShapeDtypeStruct((_B, 1), jnp.float32),
    )(deep_in, widev, w1, b1, w2, b2, w3, b3, woutt, bias)


def kernel(indices, embed_table, wide_table, wide_b, W1, b1, W2, b2, W3, b3,
           Wout, bout):
    offsets = (jnp.arange(_F, dtype=jnp.int32) * _VOCAB)[None, :]
    flat_idx = (indices.astype(jnp.int32) + offsets).reshape(
        _IDX_TOTAL // _CHUNK, _CHUNK)
    emb_flat, wide_flat = _sc_gather(flat_idx, embed_table, wide_table)
    deep_in = emb_flat.reshape(_B, _F * _DIM)
    widev = wide_flat.reshape(_B, _F)
    bias = (wide_b + bout).reshape(1, 1)
    return _mlp(deep_in, widev,
                W1, b1.reshape(1, _L1),
                W2, b2.reshape(1, _L2),
                W3, b3.reshape(1, _L3),
                Wout.reshape(1, _L3), bias)


# speedup vs baseline: 1.5783x; 1.5783x over previous
import functools

import jax
import jax.numpy as jnp
from jax import lax
from jax.experimental import pallas as pl
from jax.experimental.pallas import tpu as pltpu
from jax.experimental.pallas import tpu_sc as plsc

_B = 4096
_F = 26
_VOCAB = 100000
_DIM = 16
_L1, _L2, _L3 = 512, 256, 128

_NC = 2
_NS = 16
_NW = _NC * _NS
_IDX_TOTAL = _B * _F
_IDX_PER_W = _IDX_TOTAL // _NW
_CHUNK = 128
_NCHUNK = _IDX_PER_W // _CHUNK
_WROWS = _F * _VOCAB // _DIM


def _sc_body(idx_hbm, widx_hbm, embed_hbm, wide2d_hbm, emb_out, wslab_out,
             idx_v, widx_v, emb_v, wslab_v, sem_e, sem_w):
    wid = lax.axis_index("s") * _NC + lax.axis_index("c")
    base = wid * _IDX_PER_W
    pltpu.sync_copy(idx_hbm.at[wid], idx_v)
    pltpu.sync_copy(widx_hbm.at[wid], widx_v)

    def fire(j, carry):
        pltpu.async_copy(embed_hbm.at[idx_v.at[j]],
                         emb_v.at[pl.ds(j * _CHUNK, _CHUNK)], sem_e)
        pltpu.async_copy(wide2d_hbm.at[widx_v.at[j]],
                         wslab_v.at[pl.ds(j * _CHUNK, _CHUNK)], sem_w)
        return carry

    lax.fori_loop(0, _NCHUNK, fire, 0)

    def drain(j, carry):
        pltpu.make_async_copy(embed_hbm.at[idx_v.at[j]],
                              emb_v.at[pl.ds(j * _CHUNK, _CHUNK)], sem_e).wait()
        pltpu.make_async_copy(wide2d_hbm.at[widx_v.at[j]],
                              wslab_v.at[pl.ds(j * _CHUNK, _CHUNK)], sem_w).wait()
        return carry

    lax.fori_loop(0, _NCHUNK, drain, 0)

    pltpu.sync_copy(emb_v, emb_out.at[pl.ds(base, _IDX_PER_W)])
    pltpu.sync_copy(wslab_v, wslab_out.at[pl.ds(base, _IDX_PER_W)])


@functools.cache
def _sc_gather():
    return pl.kernel(
        _sc_body,
        out_type=[
            jax.ShapeDtypeStruct((_IDX_TOTAL, _DIM), jnp.float32),
            jax.ShapeDtypeStruct((_IDX_TOTAL, _DIM), jnp.float32),
        ],
        mesh=plsc.VectorSubcoreMesh(core_axis_name="c", subcore_axis_name="s"),
        scratch_types=[
            pltpu.VMEM((_NCHUNK, _CHUNK), jnp.int32),
            pltpu.VMEM((_NCHUNK, _CHUNK), jnp.int32),
            pltpu.VMEM((_IDX_PER_W, _DIM), jnp.float32),
            pltpu.VMEM((_IDX_PER_W, _DIM), jnp.float32),
            pltpu.SemaphoreType.DMA,
            pltpu.SemaphoreType.DMA,
        ],
        compiler_params=pltpu.CompilerParams(use_tc_tiling_on_sc=False),
    )


_BLK = 512


def _mlp_body(deep_ref, wslab_ref, col_ref, w1_ref, b1_ref, w2_ref, b2_ref,
              w3_ref, b3_ref, woutt_ref, bias_ref, out_ref):
    x = deep_ref[...]
    h = jnp.maximum(jnp.dot(x, w1_ref[...], preferred_element_type=jnp.float32)
                    + b1_ref[...], 0.0)
    h = jnp.maximum(jnp.dot(h, w2_ref[...], preferred_element_type=jnp.float32)
                    + b2_ref[...], 0.0)
    h = jnp.maximum(jnp.dot(h, w3_ref[...], preferred_element_type=jnp.float32)
                    + b3_ref[...], 0.0)
    deep = jnp.sum(h * woutt_ref[...], axis=1, keepdims=True)
    lane = jax.lax.broadcasted_iota(jnp.int32, (_BLK, _DIM), 1)
    acc = jnp.zeros((_BLK, _DIM), jnp.float32)
    for f in range(_F):
        slab_f = wslab_ref[:, f * _DIM:(f + 1) * _DIM]
        sel = (col_ref[:, f:f + 1] == lane).astype(jnp.float32)
        acc = acc + slab_f * sel
    wide = jnp.sum(acc, axis=1, keepdims=True)
    out_ref[...] = deep + wide + bias_ref[0, 0]


def _mlp(deep_in, wslab, col, w1, b1, w2, b2, w3, b3, woutt, bias):
    return pl.pallas_call(
        _mlp_body,
        grid=(_B // _BLK,),
        in_specs=[
            pl.BlockSpec((_BLK, _F * _DIM), lambda i: (i, 0)),
            pl.BlockSpec((_BLK, _F * _DIM), lambda i: (i, 0)),
            pl.BlockSpec((_BLK, _F), lambda i: (i, 0)),
            pl.BlockSpec((_F * _DIM, _L1), lambda i: (0, 0)),
            pl.BlockSpec((1, _L1), lambda i: (0, 0)),
            pl.BlockSpec((_L1, _L2), lambda i: (0, 0)),
            pl.BlockSpec((1, _L2), lambda i: (0, 0)),
            pl.BlockSpec((_L2, _L3), lambda i: (0, 0)),
            pl.BlockSpec((1, _L3), lambda i: (0, 0)),
            pl.BlockSpec((1, _L3), lambda i: (0, 0)),
            pl.BlockSpec((1, 1), lambda i: (0, 0)),
        ],
        out_specs=pl.BlockSpec((_BLK, 1), lambda i: (i, 0)),
        out_shape=jax.ShapeDtypeStruct((_B, 1), jnp.float32),
    )(deep_in, wslab, col, w1, b1, w2, b2, w3, b3, woutt, bias)


def kernel(indices, embed_table, wide_table, wide_b, W1, b1, W2, b2, W3, b3,
           Wout, bout):
    offsets = (jnp.arange(_F, dtype=jnp.int32) * _VOCAB)[None, :]
    flat_idx = indices.astype(jnp.int32) + offsets
    idx3 = flat_idx.reshape(_NW, _NCHUNK, _CHUNK)
    widx3 = (flat_idx >> 4).reshape(_NW, _NCHUNK, _CHUNK)
    col = flat_idx & 15
    wide2d = wide_table.reshape(_WROWS, _DIM)
    emb_flat, wslab_flat = _sc_gather()(idx3, widx3, embed_table, wide2d)
    deep_in = emb_flat.reshape(_B, _F * _DIM)
    wslab = wslab_flat.reshape(_B, _F * _DIM)
    bias = (wide_b + bout).reshape(1, 1)
    return _mlp(deep_in, wslab, col,
                W1, b1.reshape(1, _L1),
                W2, b2.reshape(1, _L2),
                W3, b3.reshape(1, _L3),
                Wout.reshape(1, _L3), bias)
